# baseline (speedup 1.0000x reference)
import functools

import jax
import jax.numpy as jnp
from jax import lax
from jax.experimental import pallas as pl
from jax.experimental.pallas import tpu as pltpu
from jax.experimental.pallas import tpu_sc as plsc

N_ENTITY = 1000000
N_RELATION = 1000
DIM = 128
HALF = DIM // 2
BATCH = 16384

NUM_CORES = 2
NUM_SUBCORES = 16
NUM_WORKERS = NUM_CORES * NUM_SUBCORES
B_PER_W = BATCH // NUM_WORKERS
CHUNK = 64
N_CHUNKS = B_PER_W // CHUNK
NBUF = 5
LANES = 16
UNROLL = 4


def _score_body(h_hbm, r_hbm, t_hbm, ent_hbm, rel_hbm, out_hbm,
                idxh, idxr, idxt, heads, rels, tails,
                out_v, sem_i, sems):
    wid = lax.axis_index("c") * NUM_SUBCORES + lax.axis_index("s")
    base = wid * B_PER_W

    idx_cps = []
    for c in range(N_CHUNKS):
        off = base + c * CHUNK
        idx_cps.append(pltpu.async_copy(h_hbm.at[pl.ds(off, CHUNK)], idxh.at[c], sem_i))
        idx_cps.append(pltpu.async_copy(r_hbm.at[pl.ds(off, CHUNK)], idxr.at[c], sem_i))
        idx_cps.append(pltpu.async_copy(t_hbm.at[pl.ds(off, CHUNK)], idxt.at[c], sem_i))
    for cp in idx_cps:
        cp.wait()

    def fire(c):
        b = c % NBUF
        s = sems[b]
        return (
            pltpu.async_copy(ent_hbm.at[idxh.at[c]], heads[b], s),
            pltpu.async_copy(rel_hbm.at[idxr.at[c]], rels[b], s),
            pltpu.async_copy(ent_hbm.at[idxt.at[c]], tails[b], s),
        )

    inflight = {c: fire(c) for c in range(NBUF - 1)}
    for c in range(N_CHUNKS):
        if c + NBUF - 1 < N_CHUNKS:
            inflight[c + NBUF - 1] = fire(c + NBUF - 1)
        for cp in inflight.pop(c):
            cp.wait()
        head, rel, tail = heads[c % NBUF], rels[c % NBUF], tails[c % NBUF]

        for gi in range(CHUNK // LANES):
            rows = lax.iota(jnp.int32, LANES) + (gi * LANES)

            def dim_step(q, acc):
                for u in range(UNROLL):
                    g = q * UNROLL + u
                    cr = (jnp.full((LANES,), g, jnp.int32)
                          + lax.iota(jnp.int32, LANES)) & (HALF - 1)
                    ci = cr + HALF
                    hr = plsc.load_gather(head, [rows, cr])
                    hi = plsc.load_gather(head, [rows, ci])
                    rr = plsc.load_gather(rel, [rows, cr])
                    ri = plsc.load_gather(rel, [rows, ci])
                    tr = plsc.load_gather(tail, [rows, cr])
                    ti = plsc.load_gather(tail, [rows, ci])
                    acc = acc + (hr * (rr * tr + ri * ti)
                                 + hi * (rr * ti - ri * tr))
                return acc

            acc = lax.fori_loop(0, HALF // UNROLL, dim_step,
                                jnp.zeros((LANES,), jnp.float32))
            out_v[pl.ds(c * CHUNK + gi * LANES, LANES)] = acc

    pltpu.sync_copy(out_v, out_hbm.at[pl.ds(base, B_PER_W)])


def _body_wrapper(h_hbm, r_hbm, t_hbm, ent_hbm, rel_hbm, out_hbm, *scratch):
    idxh, idxr, idxt = scratch[0:3]
    heads = scratch[3:3 + NBUF]
    rels = scratch[3 + NBUF:3 + 2 * NBUF]
    tails = scratch[3 + 2 * NBUF:3 + 3 * NBUF]
    out_v = scratch[3 + 3 * NBUF]
    sem_i = scratch[4 + 3 * NBUF]
    sems = scratch[5 + 3 * NBUF:]
    _score_body(h_hbm, r_hbm, t_hbm, ent_hbm, rel_hbm, out_hbm,
                idxh, idxr, idxt, heads, rels, tails, out_v, sem_i, sems)


@jax.jit
def _scores(h, r, t, entity_embedding, relation_embedding):
    mesh = plsc.VectorSubcoreMesh(core_axis_name="c", subcore_axis_name="s")
    k = functools.partial(
        pl.kernel,
        mesh=mesh,
        compiler_params=pltpu.CompilerParams(
            needs_layout_passes=False,
            disable_bounds_checks=True,
            disable_semaphore_checks=True,
            skip_device_barrier=True,
        ),
        out_type=jax.ShapeDtypeStruct((BATCH,), jnp.float32),
        scratch_types=[
            pltpu.VMEM((N_CHUNKS, CHUNK), jnp.int32),
            pltpu.VMEM((N_CHUNKS, CHUNK), jnp.int32),
            pltpu.VMEM((N_CHUNKS, CHUNK), jnp.int32),
            *([pltpu.VMEM((CHUNK, DIM), jnp.float32)] * (3 * NBUF)),
            pltpu.VMEM((B_PER_W,), jnp.float32),
            *([pltpu.SemaphoreType.DMA] * (1 + NBUF)),
        ],
    )(_body_wrapper)
    return k(h, r, t, entity_embedding, relation_embedding)


def kernel(h, r, t, entity_embedding, relation_embedding):
    scores = _scores(h.astype(jnp.int32), r.astype(jnp.int32),
                     t.astype(jnp.int32), entity_embedding, relation_embedding)
    return scores.reshape(BATCH, 1)


# speedup vs baseline: 1.0336x; 1.0336x over previous
import functools

import jax
import jax.numpy as jnp
from jax import lax
from jax.experimental import pallas as pl
from jax.experimental.pallas import tpu as pltpu
from jax.experimental.pallas import tpu_sc as plsc

N_ENTITY = 1000000
N_RELATION = 1000
DIM = 128
HALF = DIM // 2
BATCH = 16384

NUM_CORES = 2
NUM_SUBCORES = 16
NUM_WORKERS = NUM_CORES * NUM_SUBCORES
B_PER_W = BATCH // NUM_WORKERS
CHUNK = 64
N_CHUNKS = B_PER_W // CHUNK
NBUF = 4
LANES = 16
UNROLL = 1


def _score_body(h_hbm, r_hbm, t_hbm, ent_hbm, rel_hbm, out_hbm,
                idxh, idxr, idxt, heads, rels, tails,
                out_v, sem_i, sems):
    wid = lax.axis_index("c") * NUM_SUBCORES + lax.axis_index("s")
    base = wid * B_PER_W

    idx_cps = []
    for c in range(N_CHUNKS):
        off = base + c * CHUNK
        idx_cps.append(pltpu.async_copy(h_hbm.at[pl.ds(off, CHUNK)], idxh.at[c], sem_i))
        idx_cps.append(pltpu.async_copy(r_hbm.at[pl.ds(off, CHUNK)], idxr.at[c], sem_i))
        idx_cps.append(pltpu.async_copy(t_hbm.at[pl.ds(off, CHUNK)], idxt.at[c], sem_i))
    for cp in idx_cps:
        cp.wait()

    def fire(c):
        b = c % NBUF
        s = sems[b]
        return (
            pltpu.async_copy(ent_hbm.at[idxh.at[c]], heads[b], s),
            pltpu.async_copy(rel_hbm.at[idxr.at[c]], rels[b], s),
            pltpu.async_copy(ent_hbm.at[idxt.at[c]], tails[b], s),
        )

    inflight = {c: fire(c) for c in range(NBUF - 1)}
    for c in range(N_CHUNKS):
        if c + NBUF - 1 < N_CHUNKS:
            inflight[c + NBUF - 1] = fire(c + NBUF - 1)
        for cp in inflight.pop(c):
            cp.wait()
        head, rel, tail = heads[c % NBUF], rels[c % NBUF], tails[c % NBUF]

        for gi in range(CHUNK // LANES):
            rows = lax.iota(jnp.int32, LANES) + (gi * LANES)

            def dim_step(q, acc):
                for u in range(UNROLL):
                    g = q * UNROLL + u
                    cr = (jnp.full((LANES,), g, jnp.int32)
                          + lax.iota(jnp.int32, LANES)) & (HALF - 1)
                    ci = cr + HALF
                    hr = plsc.load_gather(head, [rows, cr])
                    hi = plsc.load_gather(head, [rows, ci])
                    rr = plsc.load_gather(rel, [rows, cr])
                    ri = plsc.load_gather(rel, [rows, ci])
                    tr = plsc.load_gather(tail, [rows, cr])
                    ti = plsc.load_gather(tail, [rows, ci])
                    acc = acc + (hr * (rr * tr + ri * ti)
                                 + hi * (rr * ti - ri * tr))
                return acc

            acc = lax.fori_loop(0, HALF // UNROLL, dim_step,
                                jnp.zeros((LANES,), jnp.float32))
            out_v[pl.ds(c * CHUNK + gi * LANES, LANES)] = acc

    pltpu.sync_copy(out_v, out_hbm.at[pl.ds(base, B_PER_W)])


def _body_wrapper(h_hbm, r_hbm, t_hbm, ent_hbm, rel_hbm, out_hbm, *scratch):
    idxh, idxr, idxt = scratch[0:3]
    heads = scratch[3:3 + NBUF]
    rels = scratch[3 + NBUF:3 + 2 * NBUF]
    tails = scratch[3 + 2 * NBUF:3 + 3 * NBUF]
    out_v = scratch[3 + 3 * NBUF]
    sem_i = scratch[4 + 3 * NBUF]
    sems = scratch[5 + 3 * NBUF:]
    _score_body(h_hbm, r_hbm, t_hbm, ent_hbm, rel_hbm, out_hbm,
                idxh, idxr, idxt, heads, rels, tails, out_v, sem_i, sems)


@jax.jit
def _scores(h, r, t, entity_embedding, relation_embedding):
    mesh = plsc.VectorSubcoreMesh(core_axis_name="c", subcore_axis_name="s")
    k = functools.partial(
        pl.kernel,
        mesh=mesh,
        compiler_params=pltpu.CompilerParams(
            needs_layout_passes=False,
            disable_bounds_checks=True,
            disable_semaphore_checks=True,
            skip_device_barrier=True,
        ),
        out_type=jax.ShapeDtypeStruct((BATCH,), jnp.float32),
        scratch_types=[
            pltpu.VMEM((N_CHUNKS, CHUNK), jnp.int32),
            pltpu.VMEM((N_CHUNKS, CHUNK), jnp.int32),
            pltpu.VMEM((N_CHUNKS, CHUNK), jnp.int32),
            *([pltpu.VMEM((CHUNK, DIM), jnp.float32)] * (3 * NBUF)),
            pltpu.VMEM((B_PER_W,), jnp.float32),
            *([pltpu.SemaphoreType.DMA] * (1 + NBUF)),
        ],
    )(_body_wrapper)
    return k(h, r, t, entity_embedding, relation_embedding)


def kernel(h, r, t, entity_embedding, relation_embedding):
    scores = _scores(h.astype(jnp.int32), r.astype(jnp.int32),
                     t.astype(jnp.int32), entity_embedding, relation_embedding)
    return scores.reshape(BATCH, 1)


# speedup vs baseline: 1.1130x; 1.0768x over previous
import functools

import jax
import jax.numpy as jnp
from jax import lax
from jax.experimental import pallas as pl
from jax.experimental.pallas import tpu as pltpu
from jax.experimental.pallas import tpu_sc as plsc

N_ENTITY = 1000000
N_RELATION = 1000
DIM = 128
HALF = DIM // 2
BATCH = 16384

NUM_CORES = 2
NUM_SUBCORES = 16
NUM_WORKERS = NUM_CORES * NUM_SUBCORES
B_PER_W = BATCH // NUM_WORKERS
CHUNK = 64
N_CHUNKS = B_PER_W // CHUNK
NBUF = 4
LANES = 16
UNROLL = 1


def _score_body(h_hbm, r_hbm, t_hbm, ent_hbm, rel_hbm, out_hbm,
                idxht, idxr, hts, rels,
                out_v, sem_i, sems):
    wid = lax.axis_index("c") * NUM_SUBCORES + lax.axis_index("s")
    base = wid * B_PER_W

    idx_cps = []
    for c in range(N_CHUNKS):
        off = base + c * CHUNK
        idx_cps.append(pltpu.async_copy(h_hbm.at[pl.ds(off, CHUNK)],
                                        idxht.at[c, pl.ds(0, CHUNK)], sem_i))
        idx_cps.append(pltpu.async_copy(t_hbm.at[pl.ds(off, CHUNK)],
                                        idxht.at[c, pl.ds(CHUNK, CHUNK)], sem_i))
        idx_cps.append(pltpu.async_copy(r_hbm.at[pl.ds(off, CHUNK)], idxr.at[c], sem_i))
    for cp in idx_cps:
        cp.wait()

    def fire(c):
        b = c % NBUF
        s = sems[b]
        return (
            pltpu.async_copy(ent_hbm.at[idxht.at[c]], hts[b], s),
            pltpu.async_copy(rel_hbm.at[idxr.at[c]], rels[b], s),
        )

    inflight = {c: fire(c) for c in range(NBUF - 1)}
    for c in range(N_CHUNKS):
        if c + NBUF - 1 < N_CHUNKS:
            inflight[c + NBUF - 1] = fire(c + NBUF - 1)
        for cp in inflight.pop(c):
            cp.wait()
        head, rel = hts[c % NBUF], rels[c % NBUF]
        tail = head

        for gi in range(CHUNK // LANES):
            rows = lax.iota(jnp.int32, LANES) + (gi * LANES)
            rows_t = rows + CHUNK

            def dim_step(q, acc):
                for u in range(UNROLL):
                    g = q * UNROLL + u
                    cr = (jnp.full((LANES,), g, jnp.int32)
                          + lax.iota(jnp.int32, LANES)) & (HALF - 1)
                    ci = cr + HALF
                    hr = plsc.load_gather(head, [rows, cr])
                    hi = plsc.load_gather(head, [rows, ci])
                    rr = plsc.load_gather(rel, [rows, cr])
                    ri = plsc.load_gather(rel, [rows, ci])
                    tr = plsc.load_gather(tail, [rows_t, cr])
                    ti = plsc.load_gather(tail, [rows_t, ci])
                    acc = acc + (hr * (rr * tr + ri * ti)
                                 + hi * (rr * ti - ri * tr))
                return acc

            acc = lax.fori_loop(0, HALF // UNROLL, dim_step,
                                jnp.zeros((LANES,), jnp.float32))
            out_v[pl.ds(c * CHUNK + gi * LANES, LANES)] = acc

    pltpu.sync_copy(out_v, out_hbm.at[pl.ds(base, B_PER_W)])


def _body_wrapper(h_hbm, r_hbm, t_hbm, ent_hbm, rel_hbm, out_hbm, *scratch):
    idxht, idxr = scratch[0:2]
    hts = scratch[2:2 + NBUF]
    rels = scratch[2 + NBUF:2 + 2 * NBUF]
    out_v = scratch[2 + 2 * NBUF]
    sem_i = scratch[3 + 2 * NBUF]
    sems = scratch[4 + 2 * NBUF:]
    _score_body(h_hbm, r_hbm, t_hbm, ent_hbm, rel_hbm, out_hbm,
                idxht, idxr, hts, rels, out_v, sem_i, sems)


@jax.jit
def _scores(h, r, t, entity_embedding, relation_embedding):
    mesh = plsc.VectorSubcoreMesh(core_axis_name="c", subcore_axis_name="s")
    k = functools.partial(
        pl.kernel,
        mesh=mesh,
        compiler_params=pltpu.CompilerParams(
            needs_layout_passes=False,
            disable_bounds_checks=True,
            disable_semaphore_checks=True,
            skip_device_barrier=True,
        ),
        out_type=jax.ShapeDtypeStruct((BATCH,), jnp.float32),
        scratch_types=[
            pltpu.VMEM((N_CHUNKS, 2 * CHUNK), jnp.int32),
            pltpu.VMEM((N_CHUNKS, CHUNK), jnp.int32),
            *([pltpu.VMEM((2 * CHUNK, DIM), jnp.float32)] * NBUF),
            *([pltpu.VMEM((CHUNK, DIM), jnp.float32)] * NBUF),
            pltpu.VMEM((B_PER_W,), jnp.float32),
            *([pltpu.SemaphoreType.DMA] * (1 + NBUF)),
        ],
    )(_body_wrapper)
    return k(h, r, t, entity_embedding, relation_embedding)


def kernel(h, r, t, entity_embedding, relation_embedding):
    scores = _scores(h.astype(jnp.int32), r.astype(jnp.int32),
                     t.astype(jnp.int32), entity_embedding, relation_embedding)
    return scores.reshape(BATCH, 1)


# speedup vs baseline: 1.1236x; 1.0096x over previous
import functools

import jax
import jax.numpy as jnp
from jax import lax
from jax.experimental import pallas as pl
from jax.experimental.pallas import tpu as pltpu
from jax.experimental.pallas import tpu_sc as plsc

N_ENTITY = 1000000
N_RELATION = 1000
DIM = 128
HALF = DIM // 2
BATCH = 16384

NUM_CORES = 2
NUM_SUBCORES = 16
NUM_WORKERS = NUM_CORES * NUM_SUBCORES
B_PER_W = BATCH // NUM_WORKERS
CHUNK = 64
N_CHUNKS = B_PER_W // CHUNK
NBUF = 4
LANES = 16
UNROLL = 1


def _score_body(h_hbm, r_hbm, t_hbm, ent_hbm, rel_hbm, out_hbm,
                idxht, idxr, hts, rels,
                out_v, sem_i, sems, sems_r):
    wid = lax.axis_index("c") * NUM_SUBCORES + lax.axis_index("s")
    base = wid * B_PER_W

    idx_cps = []
    for c in range(N_CHUNKS):
        off = base + c * CHUNK
        idx_cps.append(pltpu.async_copy(h_hbm.at[pl.ds(off, CHUNK)],
                                        idxht.at[c, pl.ds(0, CHUNK)], sem_i))
        idx_cps.append(pltpu.async_copy(t_hbm.at[pl.ds(off, CHUNK)],
                                        idxht.at[c, pl.ds(CHUNK, CHUNK)], sem_i))
    for sc in range(N_CHUNKS // 2):
        idx_cps.append(pltpu.async_copy(
            r_hbm.at[pl.ds(base + sc * 2 * CHUNK, 2 * CHUNK)], idxr.at[sc], sem_i))
    for cp in idx_cps:
        cp.wait()

    def fire(c):
        b = c % NBUF
        return pltpu.async_copy(ent_hbm.at[idxht.at[c]], hts[b], sems[b])

    def fire_rel(s):
        return pltpu.async_copy(rel_hbm.at[idxr.at[s]], rels[s % 2], sems_r[s % 2])

    inflight = {c: fire(c) for c in range(NBUF - 1)}
    relflight = {s: fire_rel(s) for s in range(min(2, N_CHUNKS // 2))}
    for c in range(N_CHUNKS):
        sc = c // 2
        if c + NBUF - 1 < N_CHUNKS:
            inflight[c + NBUF - 1] = fire(c + NBUF - 1)
        if c % 2 == 0:
            relflight.pop(sc).wait()
        inflight.pop(c).wait()
        head, rel = hts[c % NBUF], rels[sc % 2]
        tail = head

        for gi in range(CHUNK // LANES):
            rows = lax.iota(jnp.int32, LANES) + (gi * LANES)
            rows_t = rows + CHUNK
            rows_r = rows + ((c % 2) * CHUNK)

            def dim_step(q, acc):
                for u in range(UNROLL):
                    g = q * UNROLL + u
                    cr = (jnp.full((LANES,), g, jnp.int32)
                          + lax.iota(jnp.int32, LANES)) & (HALF - 1)
                    ci = cr + HALF
                    hr = plsc.load_gather(head, [rows, cr])
                    hi = plsc.load_gather(head, [rows, ci])
                    rr = plsc.load_gather(rel, [rows_r, cr])
                    ri = plsc.load_gather(rel, [rows_r, ci])
                    tr = plsc.load_gather(tail, [rows_t, cr])
                    ti = plsc.load_gather(tail, [rows_t, ci])
                    acc = acc + (hr * (rr * tr + ri * ti)
                                 + hi * (rr * ti - ri * tr))
                return acc

            acc = lax.fori_loop(0, HALF // UNROLL, dim_step,
                                jnp.zeros((LANES,), jnp.float32))
            out_v[pl.ds(c * CHUNK + gi * LANES, LANES)] = acc

        if c % 2 == 1 and sc + 2 < N_CHUNKS // 2:
            relflight[sc + 2] = fire_rel(sc + 2)

    pltpu.sync_copy(out_v, out_hbm.at[pl.ds(base, B_PER_W)])


def _body_wrapper(h_hbm, r_hbm, t_hbm, ent_hbm, rel_hbm, out_hbm, *scratch):
    idxht, idxr = scratch[0:2]
    hts = scratch[2:2 + NBUF]
    rels = scratch[2 + NBUF:4 + NBUF]
    out_v = scratch[4 + NBUF]
    sem_i = scratch[5 + NBUF]
    sems = scratch[6 + NBUF:6 + 2 * NBUF]
    sems_r = scratch[6 + 2 * NBUF:]
    _score_body(h_hbm, r_hbm, t_hbm, ent_hbm, rel_hbm, out_hbm,
                idxht, idxr, hts, rels, out_v, sem_i, sems, sems_r)


@jax.jit
def _scores(h, r, t, entity_embedding, relation_embedding):
    mesh = plsc.VectorSubcoreMesh(core_axis_name="c", subcore_axis_name="s")
    k = functools.partial(
        pl.kernel,
        mesh=mesh,
        compiler_params=pltpu.CompilerParams(
            needs_layout_passes=False,
            disable_bounds_checks=True,
            disable_semaphore_checks=True,
            skip_device_barrier=True,
        ),
        out_type=jax.ShapeDtypeStruct((BATCH,), jnp.float32),
        scratch_types=[
            pltpu.VMEM((N_CHUNKS, 2 * CHUNK), jnp.int32),
            pltpu.VMEM((N_CHUNKS // 2, 2 * CHUNK), jnp.int32),
            *([pltpu.VMEM((2 * CHUNK, DIM), jnp.float32)] * NBUF),
            *([pltpu.VMEM((2 * CHUNK, DIM), jnp.float32)] * 2),
            pltpu.VMEM((B_PER_W,), jnp.float32),
            *([pltpu.SemaphoreType.DMA] * (3 + NBUF)),
        ],
    )(_body_wrapper)
    return k(h, r, t, entity_embedding, relation_embedding)


def kernel(h, r, t, entity_embedding, relation_embedding):
    scores = _scores(h.astype(jnp.int32), r.astype(jnp.int32),
                     t.astype(jnp.int32), entity_embedding, relation_embedding)
    return scores.reshape(BATCH, 1)
